# Initial kernel scaffold; baseline (speedup 1.0000x reference)
import jax
import jax.numpy as jnp
from jax.experimental import pallas as pl


def kernel(x, edge_index, edge_weight, W_self, b_self, W_neigh, b_neigh):
    raise NotImplementedError("write your pallas kernel here")


# speedup vs baseline: 4.4744x; 4.4744x over previous
import functools

import jax
import jax.numpy as jnp
from jax import lax
from jax.experimental import pallas as pl
from jax.experimental.pallas import tpu as pltpu
from jax.experimental.pallas import tpu_sc as plsc

N = 10000
E = 320000
D = 128

NUM_CORES = 2
NUM_SUBCORES = 16
NUM_TILES = NUM_CORES * NUM_SUBCORES
EDGES_PER_TILE = E // NUM_TILES
CHUNK = 80
NUM_CHUNKS = EDGES_PER_TILE // CHUNK
ROWS_PER_SUBCORE = 624
TAIL_START = NUM_SUBCORES * ROWS_PER_SUBCORE
TAIL_ROWS = N - TAIL_START


def _sc_segment_sum(x, src, dst, w, zeros):
    mesh = plsc.VectorSubcoreMesh(core_axis_name="c", subcore_axis_name="s")

    @functools.partial(
        pl.kernel,
        out_type=jax.ShapeDtypeStruct((NUM_CORES, N, D), jnp.float32),
        mesh=mesh,
        scratch_types=[
            pltpu.VMEM((CHUNK,), jnp.int32),
            pltpu.VMEM((CHUNK,), jnp.int32),
            pltpu.VMEM((CHUNK,), jnp.float32),
            pltpu.VMEM((CHUNK, D), jnp.float32),
            pltpu.VMEM_SHARED((N, D), jnp.float32),
            pltpu.SemaphoreType.DMA,
        ],
    )
    def sc_kernel(x_hbm, src_hbm, dst_hbm, w_hbm, zeros_hbm, out_hbm,
                  idx_s, idx_d, w_v, rows_v, acc_sh, sem):
        cid = lax.axis_index("c")
        sid = lax.axis_index("s")
        tile = cid * NUM_SUBCORES + sid

        row0 = sid * ROWS_PER_SUBCORE
        pltpu.sync_copy(zeros_hbm, acc_sh.at[pl.ds(row0, ROWS_PER_SUBCORE)])

        @pl.when(sid == NUM_SUBCORES - 1)
        def _zero_tail():
            pltpu.sync_copy(zeros_hbm.at[pl.ds(0, TAIL_ROWS)],
                            acc_sh.at[pl.ds(TAIL_START, TAIL_ROWS)])

        plsc.subcore_barrier()

        base0 = tile * EDGES_PER_TILE

        def chunk_body(ch, carry):
            base = base0 + ch * CHUNK
            pltpu.sync_copy(src_hbm.at[pl.ds(base, CHUNK)], idx_s)
            pltpu.sync_copy(dst_hbm.at[pl.ds(base, CHUNK)], idx_d)
            pltpu.sync_copy(w_hbm.at[pl.ds(base, CHUNK)], w_v)
            pltpu.async_copy(x_hbm.at[idx_s], rows_v, sem).wait()

            def group_body(g, c2):
                wvec = w_v[pl.ds(g * 16, 16)]
                for j in range(16):
                    wv = wvec[j]
                    e = g * 16 + j
                    for cc in range(D // 16):
                        sl = pl.ds(cc * 16, 16)
                        rows_v[e, sl] = rows_v[e, sl] * wv
                return c2

            lax.fori_loop(0, CHUNK // 16, group_body, 0)

            pltpu.sync_copy(rows_v, acc_sh.at[idx_d], add=True)
            return carry

        lax.fori_loop(0, NUM_CHUNKS, chunk_body, 0)
        plsc.subcore_barrier()

        pltpu.sync_copy(acc_sh.at[pl.ds(row0, ROWS_PER_SUBCORE)],
                        out_hbm.at[cid, pl.ds(row0, ROWS_PER_SUBCORE)])

        @pl.when(sid == NUM_SUBCORES - 1)
        def _dump_tail():
            pltpu.sync_copy(acc_sh.at[pl.ds(TAIL_START, TAIL_ROWS)],
                            out_hbm.at[cid, pl.ds(TAIL_START, TAIL_ROWS)])

    return sc_kernel(x, src, dst, w, zeros)


def _tc_dense_body(x_r, p_r, ws_r, wn_r, bs_r, bn_r, o_r):
    acc = lax.dot_general(x_r[...], ws_r[...], (((1,), (1,)), ((), ())),
                          preferred_element_type=jnp.float32)
    neigh = p_r[0] + p_r[1]
    acc = acc + lax.dot_general(neigh, wn_r[...], (((1,), (1,)), ((), ())),
                                preferred_element_type=jnp.float32)
    o_r[...] = jnp.maximum(acc + (bs_r[...] + bn_r[...])[None, :], 0.0)


def _tc_dense(x, partials, W_self, b_self, W_neigh, b_neigh):
    R = 1000
    grid = (N // R,)
    return pl.pallas_call(
        _tc_dense_body,
        grid=grid,
        in_specs=[
            pl.BlockSpec((R, D), lambda i: (i, 0)),
            pl.BlockSpec((NUM_CORES, R, D), lambda i: (0, i, 0)),
            pl.BlockSpec((D, D), lambda i: (0, 0)),
            pl.BlockSpec((D, D), lambda i: (0, 0)),
            pl.BlockSpec((D,), lambda i: (0,)),
            pl.BlockSpec((D,), lambda i: (0,)),
        ],
        out_specs=pl.BlockSpec((R, D), lambda i: (i, 0)),
        out_shape=jax.ShapeDtypeStruct((N, D), jnp.float32),
    )(x, partials, W_self, W_neigh, b_self, b_neigh)


def kernel(x, edge_index, edge_weight, W_self, b_self, W_neigh, b_neigh):
    dst = edge_index[0].astype(jnp.int32)
    src = edge_index[1].astype(jnp.int32)
    zeros = jnp.zeros((ROWS_PER_SUBCORE, D), jnp.float32)
    partials = _sc_segment_sum(x, src, dst, edge_weight, zeros)
    return _tc_dense(x, partials, W_self, b_self, W_neigh, b_neigh)


# speedup vs baseline: 8.6339x; 1.9296x over previous
import functools

import jax
import jax.numpy as jnp
from jax import lax
from jax.experimental import pallas as pl
from jax.experimental.pallas import tpu as pltpu
from jax.experimental.pallas import tpu_sc as plsc

N = 10000
E = 320000
D = 128

NUM_CORES = 2
NUM_SUBCORES = 16
NUM_TILES = NUM_CORES * NUM_SUBCORES
EDGES_PER_TILE = E // NUM_TILES
CHUNK = 80
NUM_CHUNKS = EDGES_PER_TILE // CHUNK
NUM_PAIRS = NUM_CHUNKS // 2
ROWS_PER_SUBCORE = 624
TAIL_START = NUM_SUBCORES * ROWS_PER_SUBCORE
TAIL_ROWS = N - TAIL_START


def _sc_segment_sum(x, src, dst, w, zeros):
    mesh = plsc.VectorSubcoreMesh(core_axis_name="c", subcore_axis_name="s")

    @functools.partial(
        pl.kernel,
        out_type=jax.ShapeDtypeStruct((NUM_CORES, N, D), jnp.float32),
        mesh=mesh,
        scratch_types=[
            pltpu.VMEM((CHUNK,), jnp.int32),
            pltpu.VMEM((CHUNK,), jnp.int32),
            pltpu.VMEM((CHUNK,), jnp.float32),
            pltpu.VMEM((CHUNK,), jnp.int32),
            pltpu.VMEM((CHUNK,), jnp.int32),
            pltpu.VMEM((CHUNK,), jnp.float32),
            pltpu.VMEM((CHUNK, D), jnp.float32),
            pltpu.VMEM((CHUNK, D), jnp.float32),
            pltpu.VMEM_SHARED((N, D), jnp.float32),
            pltpu.SemaphoreType.DMA,
            pltpu.SemaphoreType.DMA,
            pltpu.SemaphoreType.DMA,
            pltpu.SemaphoreType.DMA,
        ],
    )
    def sc_kernel(x_hbm, src_hbm, dst_hbm, w_hbm, zeros_hbm, out_hbm,
                  is0, id0, w0, is1, id1, w1, rows0, rows1, acc_sh,
                  sem_s0, sem_s1, sem_g0, sem_g1):
        cid = lax.axis_index("c")
        sid = lax.axis_index("s")
        tile = cid * NUM_SUBCORES + sid
        base0 = tile * EDGES_PER_TILE

        row0 = sid * ROWS_PER_SUBCORE
        pltpu.sync_copy(zeros_hbm, acc_sh.at[pl.ds(row0, ROWS_PER_SUBCORE)])

        @pl.when(sid == NUM_SUBCORES - 1)
        def _zero_tail():
            pltpu.sync_copy(zeros_hbm.at[pl.ds(0, TAIL_ROWS)],
                            acc_sh.at[pl.ds(TAIL_START, TAIL_ROWS)])

        plsc.subcore_barrier()

        def stage_fire(c, i_s, i_d, w_v, sem):
            base = base0 + c * CHUNK
            pltpu.async_copy(src_hbm.at[pl.ds(base, CHUNK)], i_s, sem)
            pltpu.async_copy(dst_hbm.at[pl.ds(base, CHUNK)], i_d, sem)
            pltpu.async_copy(w_hbm.at[pl.ds(base, CHUNK)], w_v, sem)

        def stage_drain(c, i_s, i_d, w_v, sem):
            base = base0 + c * CHUNK
            pltpu.make_async_copy(src_hbm.at[pl.ds(base, CHUNK)], i_s, sem).wait()
            pltpu.make_async_copy(dst_hbm.at[pl.ds(base, CHUNK)], i_d, sem).wait()
            pltpu.make_async_copy(w_hbm.at[pl.ds(base, CHUNK)], w_v, sem).wait()

        def gather_fire(i_s, rows_v, sem):
            pltpu.async_copy(x_hbm.at[i_s], rows_v, sem)

        def gather_drain(i_s, rows_v, sem):
            pltpu.make_async_copy(x_hbm.at[i_s], rows_v, sem).wait()

        def scale(rows_v, w_v):
            def group_body(g, c2):
                wvec = w_v[pl.ds(g * 16, 16)]
                for j in range(16):
                    wv = wvec[j]
                    e = g * 16 + j
                    for cc in range(D // 16):
                        sl = pl.ds(cc * 16, 16)
                        rows_v[e, sl] = rows_v[e, sl] * wv
                return c2

            lax.fori_loop(0, CHUNK // 16, group_body, 0)

        def process(rows_v, w_v, i_d):
            scale(rows_v, w_v)
            pltpu.sync_copy(rows_v, acc_sh.at[i_d], add=True)

        stage_fire(0, is0, id0, w0, sem_s0)
        stage_fire(1, is1, id1, w1, sem_s1)
        stage_drain(0, is0, id0, w0, sem_s0)
        gather_fire(is0, rows0, sem_g0)

        def pair_body(g, carry):
            c0 = 2 * g
            stage_drain(c0 + 1, is1, id1, w1, sem_s1)
            gather_fire(is1, rows1, sem_g1)
            gather_drain(is0, rows0, sem_g0)
            process(rows0, w0, id0)
            stage_fire(c0 + 2, is0, id0, w0, sem_s0)
            gather_drain(is1, rows1, sem_g1)
            process(rows1, w1, id1)
            stage_drain(c0 + 2, is0, id0, w0, sem_s0)
            gather_fire(is0, rows0, sem_g0)

            @pl.when(c0 + 3 < NUM_CHUNKS)
            def _stage_next():
                stage_fire(c0 + 3, is1, id1, w1, sem_s1)

            return carry

        lax.fori_loop(0, NUM_PAIRS, pair_body, 0)

        gather_drain(is0, rows0, sem_g0)
        process(rows0, w0, id0)

        plsc.subcore_barrier()

        pltpu.sync_copy(acc_sh.at[pl.ds(row0, ROWS_PER_SUBCORE)],
                        out_hbm.at[cid, pl.ds(row0, ROWS_PER_SUBCORE)])

        @pl.when(sid == NUM_SUBCORES - 1)
        def _dump_tail():
            pltpu.sync_copy(acc_sh.at[pl.ds(TAIL_START, TAIL_ROWS)],
                            out_hbm.at[cid, pl.ds(TAIL_START, TAIL_ROWS)])

    return sc_kernel(x, src, dst, w, zeros)


def _tc_dense_body(x_r, p_r, ws_r, wn_r, bs_r, bn_r, o_r):
    acc = lax.dot_general(x_r[...], ws_r[...], (((1,), (1,)), ((), ())),
                          preferred_element_type=jnp.float32)
    neigh = p_r[0] + p_r[1]
    acc = acc + lax.dot_general(neigh, wn_r[...], (((1,), (1,)), ((), ())),
                                preferred_element_type=jnp.float32)
    o_r[...] = jnp.maximum(acc + (bs_r[...] + bn_r[...])[None, :], 0.0)


def _tc_dense(x, partials, W_self, b_self, W_neigh, b_neigh):
    R = 1000
    grid = (N // R,)
    return pl.pallas_call(
        _tc_dense_body,
        grid=grid,
        in_specs=[
            pl.BlockSpec((R, D), lambda i: (i, 0)),
            pl.BlockSpec((NUM_CORES, R, D), lambda i: (0, i, 0)),
            pl.BlockSpec((D, D), lambda i: (0, 0)),
            pl.BlockSpec((D, D), lambda i: (0, 0)),
            pl.BlockSpec((D,), lambda i: (0,)),
            pl.BlockSpec((D,), lambda i: (0,)),
        ],
        out_specs=pl.BlockSpec((R, D), lambda i: (i, 0)),
        out_shape=jax.ShapeDtypeStruct((N, D), jnp.float32),
    )(x, partials, W_self, W_neigh, b_self, b_neigh)


def kernel(x, edge_index, edge_weight, W_self, b_self, W_neigh, b_neigh):
    dst = edge_index[0].astype(jnp.int32)
    src = edge_index[1].astype(jnp.int32)
    zeros = jnp.zeros((ROWS_PER_SUBCORE, D), jnp.float32)
    partials = _sc_segment_sum(x, src, dst, edge_weight, zeros)
    return _tc_dense(x, partials, W_self, b_self, W_neigh, b_neigh)
